# speedup vs baseline: 4.7400x; 4.7400x over previous
import functools

import jax
import jax.numpy as jnp
from jax import lax
from jax.experimental import pallas as pl
from jax.experimental.pallas import tpu as pltpu
from jax.experimental.pallas import tpu_sc as plsc

N_NODES = 10000
D = 128

NC = 2
NS = 16
NW = NC * NS

K = 128
N_TAB = 10240
ROWS_PER_TILE = N_TAB // NS


def _sc_body(n_chunks, x_hbm, src_hbm, dst_hbm, agg_out, deg_out,
             sidx_v, didx_v, rows_v, ones_v, degz_v, sem, agg_sh, deg_sh):
    c = lax.axis_index("c")
    s = lax.axis_index("s")
    wid = c * NS + s

    zero16 = jnp.zeros((16,), jnp.float32)
    one16 = jnp.ones((16,), jnp.float32)

    def _zrow(i, _):
        for j in range(D // 16):
            rows_v[i, pl.ds(j * 16, 16)] = zero16
        return 0
    lax.fori_loop(0, K, _zrow, 0)

    def _zdeg(i, _):
        degz_v[pl.ds(i * 16, 16)] = zero16
        return 0
    lax.fori_loop(0, ROWS_PER_TILE // 16, _zdeg, 0)

    for j in range(K // 16):
        ones_v[pl.ds(j * 16, 16)] = one16

    for bblk in range(ROWS_PER_TILE // K):
        pltpu.sync_copy(rows_v, agg_sh.at[pl.ds(s * ROWS_PER_TILE + bblk * K, K)])
    pltpu.sync_copy(degz_v, deg_sh.at[pl.ds(s * ROWS_PER_TILE, ROWS_PER_TILE)])

    plsc.subcore_barrier()

    edges_per_tile = n_chunks * K

    def _chunk(i, _):
        base = wid * edges_per_tile + i * K
        pltpu.sync_copy(src_hbm.at[pl.ds(base, K)], sidx_v)
        pltpu.sync_copy(dst_hbm.at[pl.ds(base, K)], didx_v)
        pltpu.async_copy(x_hbm.at[sidx_v], rows_v, sem).wait()
        pltpu.sync_copy(rows_v, agg_sh.at[didx_v], add=True)
        pltpu.sync_copy(ones_v, deg_sh.at[didx_v], add=True)
        return 0
    lax.fori_loop(0, n_chunks, _chunk, 0)

    plsc.subcore_barrier()

    pltpu.sync_copy(agg_sh.at[pl.ds(s * ROWS_PER_TILE, ROWS_PER_TILE)],
                    agg_out.at[c, pl.ds(s * ROWS_PER_TILE, ROWS_PER_TILE)])
    pltpu.sync_copy(deg_sh.at[pl.ds(s * ROWS_PER_TILE, ROWS_PER_TILE)],
                    deg_out.at[c, pl.ds(s * ROWS_PER_TILE, ROWS_PER_TILE)])


def _segment_mean_sc(x, src_pad, dst_pad, n_chunks):
    mesh = plsc.VectorSubcoreMesh(core_axis_name="c", subcore_axis_name="s")
    return pl.kernel(
        functools.partial(_sc_body, n_chunks),
        out_type=(
            jax.ShapeDtypeStruct((NC, N_TAB, D), jnp.float32),
            jax.ShapeDtypeStruct((NC, N_TAB), jnp.float32),
        ),
        mesh=mesh,
        scratch_types=[
            pltpu.VMEM((K,), jnp.int32),
            pltpu.VMEM((K,), jnp.int32),
            pltpu.VMEM((K, D), jnp.float32),
            pltpu.VMEM((K,), jnp.float32),
            pltpu.VMEM((ROWS_PER_TILE,), jnp.float32),
            pltpu.SemaphoreType.DMA,
            pltpu.VMEM_SHARED((N_TAB, D), jnp.float32),
            pltpu.VMEM_SHARED((N_TAB,), jnp.float32),
        ],
        name="gcn_segment_mean_sc",
    )(x, src_pad, dst_pad)


def _tc_body(agg_ref, deg_ref, w_ref, b_ref, out_ref):
    agg = agg_ref[0] + agg_ref[1]
    deg = deg_ref[0] + deg_ref[1]
    mean = agg / jnp.maximum(deg, 1.0)[:, None]
    out_ref[...] = (
        jnp.dot(mean, w_ref[...].T, preferred_element_type=jnp.float32)
        + b_ref[...]
    )


def _linear_tc(agg, deg, W, b):
    bs = 1024
    grid = (N_TAB // bs,)
    return pl.pallas_call(
        _tc_body,
        grid=grid,
        in_specs=[
            pl.BlockSpec((NC, bs, D), lambda i: (0, i, 0)),
            pl.BlockSpec((NC, bs), lambda i: (0, i)),
            pl.BlockSpec((D, D), lambda i: (0, 0)),
            pl.BlockSpec((1, D), lambda i: (0, 0)),
        ],
        out_specs=pl.BlockSpec((bs, D), lambda i: (i, 0)),
        out_shape=jax.ShapeDtypeStruct((N_TAB, D), jnp.float32),
    )(agg, deg, W, b.reshape(1, D))


def kernel(x, edge_index, W, b):
    e = edge_index.shape[1]
    src = edge_index[0]
    dst = edge_index[1]
    n_chunks = -(-e // (NW * K))
    e_pad = n_chunks * NW * K
    pad = e_pad - e
    if pad:
        src = jnp.concatenate([src, jnp.zeros((pad,), jnp.int32)])
        dst = jnp.concatenate(
            [dst, N_NODES + (jnp.arange(pad, dtype=jnp.int32) % (N_TAB - N_NODES))])
    agg, deg = _segment_mean_sc(x, src, dst, n_chunks)
    out = _linear_tc(agg, deg, W, b)
    return out[:N_NODES]
